# baseline (speedup 1.0000x reference)
import functools

import jax
import jax.numpy as jnp
from jax import lax
from jax.experimental import pallas as pl
from jax.experimental.pallas import tpu as pltpu
from jax.experimental.pallas import tpu_sc as plsc

NC = 2
NS = 16
NW = NC * NS
K = 128
CS = 8
CPT = 80


def _sc_scatter(table, src2d, dst2d, zeros, n_pad):
    d = table.shape[1]
    rows_per_tile = n_pad // NS
    nseg = CPT // CS
    mesh = plsc.VectorSubcoreMesh(core_axis_name="c", subcore_axis_name="s")

    @functools.partial(
        pl.kernel,
        mesh=mesh,
        out_type=jax.ShapeDtypeStruct((NC, n_pad, d), jnp.float32),
        scratch_types=[
            pltpu.VMEM((CS, K), jnp.int32),
            pltpu.VMEM((CS, K), jnp.int32),
            pltpu.VMEM((CS, K), jnp.int32),
            pltpu.VMEM((CS, K), jnp.int32),
            pltpu.VMEM((K, d), jnp.float32),
            pltpu.VMEM((K, d), jnp.float32),
            pltpu.SemaphoreType.DMA,
            pltpu.SemaphoreType.DMA,
            pltpu.SemaphoreType.DMA,
            pltpu.SemaphoreType.DMA,
            pltpu.VMEM_SHARED((n_pad, d), jnp.float32),
        ],
    )
    def body(table_hbm, src_hbm, dst_hbm, zeros_hbm, out_hbm,
             isa, ida, isb, idb, rows0, rows1, sem0, sem1, semia, semib, acc):
        cid = lax.axis_index("c")
        sid = lax.axis_index("s")
        wid = sid * NC + cid
        base = wid * CPT

        def gather(idx, rows, sem):
            return pltpu.async_copy(table_hbm.at[idx], rows, sem)

        def prefetch(s, i_s, i_d, semi):
            a = pltpu.async_copy(src_hbm.at[pl.ds(base + s * CS, CS)], i_s,
                                 semi)
            b = pltpu.async_copy(dst_hbm.at[pl.ds(base + s * CS, CS)], i_d,
                                 semi)
            return a, b

        def segment(i_s, i_d):
            g0 = gather(i_s.at[0], rows0, sem0)
            g1 = gather(i_s.at[1], rows1, sem1)

            def pair(p, carry):
                c = 2 * p
                g0.wait()
                pltpu.sync_copy(rows0, acc.at[i_d.at[c]], add=True)
                gather(i_s.at[c + 2], rows0, sem0)
                g1.wait()
                pltpu.sync_copy(rows1, acc.at[i_d.at[c + 1]], add=True)
                gather(i_s.at[c + 3], rows1, sem1)
                return carry

            lax.fori_loop(0, CS // 2 - 1, pair, 0)
            g0.wait()
            pltpu.sync_copy(rows0, acc.at[i_d.at[CS - 2]], add=True)
            g1.wait()
            pltpu.sync_copy(rows1, acc.at[i_d.at[CS - 1]], add=True)

        pa, pa2 = prefetch(0, isa, ida, semia)
        pb, pb2 = prefetch(1, isb, idb, semib)
        r0 = sid * rows_per_tile
        pltpu.sync_copy(zeros_hbm.at[pl.ds(r0, rows_per_tile)],
                        acc.at[pl.ds(r0, rows_per_tile)])
        plsc.subcore_barrier()

        def seg_pair(t, carry):
            s = 2 * t
            pa.wait()
            pa2.wait()
            segment(isa, ida)

            @pl.when(s + 2 < nseg)
            def _():
                prefetch(s + 2, isa, ida, semia)

            pb.wait()
            pb2.wait()
            segment(isb, idb)

            @pl.when(s + 3 < nseg)
            def _():
                prefetch(s + 3, isb, idb, semib)

            return carry

        lax.fori_loop(0, nseg // 2, seg_pair, 0)
        plsc.subcore_barrier()

        pltpu.sync_copy(acc.at[pl.ds(r0, rows_per_tile)],
                        out_hbm.at[cid, pl.ds(r0, rows_per_tile)])

    return body(table, src2d, dst2d, zeros)


def _tc_combine(node, acc, eps):
    n, d = node.shape
    blk = 400
    grid = n // blk

    def body(eps_ref, node_ref, a_ref, out_ref):
        scale = 1.0 + eps_ref[0]
        out_ref[...] = scale * node_ref[...] + a_ref[0] + a_ref[1]

    return pl.pallas_call(
        body,
        grid=(grid,),
        in_specs=[
            pl.BlockSpec(memory_space=pltpu.SMEM),
            pl.BlockSpec((blk, d), lambda i: (i, 0)),
            pl.BlockSpec((NC, blk, d), lambda i: (0, i, 0)),
        ],
        out_specs=pl.BlockSpec((blk, d), lambda i: (i, 0)),
        out_shape=jax.ShapeDtypeStruct((n, d), jnp.float32),
    )(eps, node, acc)


def kernel(node, edge_index, eps_k):
    n, d = node.shape
    m = edge_index.shape[1]
    src = edge_index[1]
    dst = edge_index[0]

    m_pad = NW * CPT * K
    n_pad = -(-(n + 1) // (NS * 8)) * (NS * 8)

    pad = m_pad - m
    src = jnp.concatenate([src, jnp.zeros((pad,), jnp.int32)])
    dst_pad = n + jnp.arange(pad, dtype=jnp.int32) % (n_pad - n)
    dst = jnp.concatenate([dst, dst_pad])
    table = jnp.concatenate([node, node])
    src = src.reshape(NW, CPT * K)
    src = src + (n * (jnp.arange(NW, dtype=jnp.int32) % 2))[:, None]
    src2d = src.reshape(m_pad // K, K)
    dst2d = dst.reshape(m_pad // K, K)

    zeros = jnp.zeros((n_pad, d), jnp.float32)
    acc = _sc_scatter(table, src2d, dst2d, zeros, n_pad)
    return _tc_combine(node, acc, jnp.reshape(eps_k, (1,)))


# speedup vs baseline: 1.0069x; 1.0069x over previous
import functools

import jax
import jax.numpy as jnp
from jax import lax
from jax.experimental import pallas as pl
from jax.experimental.pallas import tpu as pltpu
from jax.experimental.pallas import tpu_sc as plsc

NC = 2
NS = 16
NW = NC * NS
K = 128
CS = 16


def _sc_scatter(table, src2d, dst2d, zeros, n_pad, cpt):
    d = table.shape[1]
    rows_per_tile = n_pad // NS
    seg = cpt // CS
    mesh = plsc.VectorSubcoreMesh(core_axis_name="c", subcore_axis_name="s")

    @functools.partial(
        pl.kernel,
        mesh=mesh,
        out_type=jax.ShapeDtypeStruct((NC, n_pad, d), jnp.float32),
        scratch_types=[
            pltpu.VMEM((CS, K), jnp.int32),
            pltpu.VMEM((CS, K), jnp.int32),
            pltpu.VMEM((CS, K), jnp.int32),
            pltpu.VMEM((CS, K), jnp.int32),
            pltpu.VMEM((K, d), jnp.float32),
            pltpu.VMEM((K, d), jnp.float32),
            pltpu.VMEM_SHARED((n_pad, d), jnp.float32),
            pltpu.SemaphoreType.DMA,
            pltpu.SemaphoreType.DMA,
            pltpu.SemaphoreType.DMA,
        ],
    )
    def body(table_hbm, src_hbm, dst_hbm, zeros_hbm, out_hbm,
             is0, is1, id0, id1, rows0, rows1, acc, sem0, sem1, semi):
        cid = lax.axis_index("c")
        sid = lax.axis_index("s")
        wid = sid * NC + cid
        isb = (is0, is1)
        idb = (id0, id1)

        base = wid * cpt
        p0 = pltpu.async_copy(src_hbm.at[pl.ds(base, CS)], is0, semi)
        p1 = pltpu.async_copy(dst_hbm.at[pl.ds(base, CS)], id0, semi)
        r0 = sid * rows_per_tile
        pltpu.sync_copy(zeros_hbm.at[pl.ds(r0, rows_per_tile)],
                        acc.at[pl.ds(r0, rows_per_tile)])
        p0.wait()
        p1.wait()
        plsc.subcore_barrier()

        def gather(idx, rows, sem):
            return pltpu.async_copy(table_hbm.at[idx], rows, sem)

        for s in range(seg):
            i_s = isb[s % 2]
            i_d = idb[s % 2]
            g0 = gather(i_s.at[0], rows0, sem0)
            g1 = gather(i_s.at[1], rows1, sem1)
            if s + 1 < seg:
                nbase = base + (s + 1) * CS
                ps = pltpu.async_copy(src_hbm.at[pl.ds(nbase, CS)],
                                      isb[(s + 1) % 2], semi)
                pd = pltpu.async_copy(dst_hbm.at[pl.ds(nbase, CS)],
                                      idb[(s + 1) % 2], semi)

            def pair(p, carry):
                c = 2 * p
                g0.wait()
                pltpu.sync_copy(rows0, acc.at[i_d.at[c]], add=True)

                @pl.when(c + 2 < CS)
                def _():
                    gather(i_s.at[c + 2], rows0, sem0)

                g1.wait()
                pltpu.sync_copy(rows1, acc.at[i_d.at[c + 1]], add=True)

                @pl.when(c + 3 < CS)
                def _():
                    gather(i_s.at[c + 3], rows1, sem1)

                return carry

            lax.fori_loop(0, CS // 2, pair, 0)
            if s + 1 < seg:
                ps.wait()
                pd.wait()

        plsc.subcore_barrier()

        pltpu.sync_copy(acc.at[pl.ds(r0, rows_per_tile)],
                        out_hbm.at[cid, pl.ds(r0, rows_per_tile)])

    return body(table, src2d, dst2d, zeros)


def _tc_combine(node, acc, eps):
    n, d = node.shape
    blk = 400
    grid = n // blk

    def body(eps_ref, node_ref, a_ref, out_ref):
        scale = 1.0 + eps_ref[0]
        out_ref[...] = scale * node_ref[...] + a_ref[0] + a_ref[1]

    return pl.pallas_call(
        body,
        grid=(grid,),
        in_specs=[
            pl.BlockSpec(memory_space=pltpu.SMEM),
            pl.BlockSpec((blk, d), lambda i: (i, 0)),
            pl.BlockSpec((NC, blk, d), lambda i: (0, i, 0)),
        ],
        out_specs=pl.BlockSpec((blk, d), lambda i: (i, 0)),
        out_shape=jax.ShapeDtypeStruct((n, d), jnp.float32),
    )(eps, node, acc)


def kernel(node, edge_index, eps_k):
    n, d = node.shape
    m = edge_index.shape[1]
    src = edge_index[1]
    dst = edge_index[0]

    cpt = -(-m // (NW * K * CS)) * CS
    m_pad = NW * K * cpt
    n_pad = -(-(n + 1) // (NS * 8)) * (NS * 8)

    pad = m_pad - m
    src = jnp.concatenate([src, jnp.zeros((pad,), jnp.int32)])
    dst_pad = n + jnp.arange(pad, dtype=jnp.int32) % (n_pad - n)
    dst = jnp.concatenate([dst, dst_pad])
    table = jnp.concatenate([node, node])
    src = src.reshape(NW, cpt * K)
    src = src + (n * (jnp.arange(NW, dtype=jnp.int32) % 2))[:, None]
    src2d = src.reshape(m_pad // K, K)
    dst2d = dst.reshape(m_pad // K, K)

    zeros = jnp.zeros((n_pad, d), jnp.float32)
    acc = _sc_scatter(table, src2d, dst2d, zeros, n_pad, cpt)
    return _tc_combine(node, acc, jnp.reshape(eps_k, (1,)))


# speedup vs baseline: 1.2725x; 1.2638x over previous
import functools

import jax
import jax.numpy as jnp
from jax import lax
from jax.experimental import pallas as pl
from jax.experimental.pallas import tpu as pltpu
from jax.experimental.pallas import tpu_sc as plsc

NC = 2
NS = 16
NW = NC * NS
K = 128
CS = 16


def _sc_scatter(table, src2d, dst2d, zeros, n_pad, cpt):
    d = table.shape[1]
    rows_per_tile = n_pad // NS
    seg = cpt // CS
    mesh = plsc.VectorSubcoreMesh(core_axis_name="c", subcore_axis_name="s")

    @functools.partial(
        pl.kernel,
        mesh=mesh,
        out_type=jax.ShapeDtypeStruct((NC, n_pad, d), jnp.float32),
        scratch_types=[
            pltpu.VMEM((CS, K), jnp.int32),
            pltpu.VMEM((CS, K), jnp.int32),
            pltpu.VMEM((CS, K), jnp.int32),
            pltpu.VMEM((CS, K), jnp.int32),
            pltpu.VMEM((K, d), jnp.float32),
            pltpu.VMEM((K, d), jnp.float32),
            pltpu.VMEM_SHARED((n_pad, d), jnp.float32),
            pltpu.SemaphoreType.DMA,
            pltpu.SemaphoreType.DMA,
            pltpu.SemaphoreType.DMA,
        ],
    )
    def body(table_hbm, src_hbm, dst_hbm, zeros_hbm, out_hbm,
             is0, is1, id0, id1, rows0, rows1, acc, sem0, sem1, semi):
        cid = lax.axis_index("c")
        sid = lax.axis_index("s")
        wid = sid * NC + cid
        isb = (is0, is1)
        idb = (id0, id1)

        base = wid * cpt
        p0 = pltpu.async_copy(src_hbm.at[pl.ds(base, CS)], is0, semi)
        p1 = pltpu.async_copy(dst_hbm.at[pl.ds(base, CS)], id0, semi)
        r0 = sid * rows_per_tile
        pltpu.sync_copy(zeros_hbm.at[pl.ds(r0, rows_per_tile)],
                        acc.at[pl.ds(r0, rows_per_tile)])
        p0.wait()
        p1.wait()
        plsc.subcore_barrier()

        def gather(idx, rows, sem):
            return pltpu.async_copy(table_hbm.at[idx], rows, sem)

        for s in range(seg):
            i_s = isb[s % 2]
            i_d = idb[s % 2]
            g0 = gather(i_s.at[0], rows0, sem0)
            g1 = gather(i_s.at[1], rows1, sem1)
            if s + 1 < seg:
                nbase = base + (s + 1) * CS
                ps = pltpu.async_copy(src_hbm.at[pl.ds(nbase, CS)],
                                      isb[(s + 1) % 2], semi)
                pd = pltpu.async_copy(dst_hbm.at[pl.ds(nbase, CS)],
                                      idb[(s + 1) % 2], semi)

            def pair(p, carry):
                c = 2 * p
                g0.wait()
                pltpu.sync_copy(rows0, acc.at[i_d.at[c]], add=True)

                @pl.when(c + 2 < CS)
                def _():
                    gather(i_s.at[c + 2], rows0, sem0)

                g1.wait()
                pltpu.sync_copy(rows1, acc.at[i_d.at[c + 1]], add=True)

                @pl.when(c + 3 < CS)
                def _():
                    gather(i_s.at[c + 3], rows1, sem1)

                return carry

            lax.fori_loop(0, CS // 2, pair, 0)
            if s + 1 < seg:
                ps.wait()
                pd.wait()

        plsc.subcore_barrier()

        pltpu.sync_copy(acc.at[pl.ds(r0, rows_per_tile)],
                        out_hbm.at[cid, pl.ds(r0, rows_per_tile)])

    return body(table, src2d, dst2d, zeros)


def _tc_combine(node, acc0, acc1, eps):
    n, d = node.shape
    blk = 2000
    grid = n // blk

    def body(eps_ref, node_ref, a0_ref, a1_ref, out_ref):
        scale = 1.0 + eps_ref[0]
        out_ref[...] = scale * node_ref[...] + a0_ref[...] + a1_ref[...]

    return pl.pallas_call(
        body,
        grid=(grid,),
        in_specs=[
            pl.BlockSpec(memory_space=pltpu.SMEM),
            pl.BlockSpec((blk, d), lambda i: (i, 0)),
            pl.BlockSpec((blk, d), lambda i: (i, 0)),
            pl.BlockSpec((blk, d), lambda i: (i, 0)),
        ],
        out_specs=pl.BlockSpec((blk, d), lambda i: (i, 0)),
        out_shape=jax.ShapeDtypeStruct((n, d), jnp.float32),
    )(eps, node, acc0, acc1)


def kernel(node, edge_index, eps_k):
    n, d = node.shape
    m = edge_index.shape[1]
    src = edge_index[1]
    dst = edge_index[0]

    cpt = -(-m // (NW * K * CS)) * CS
    m_pad = NW * K * cpt
    n_pad = -(-(n + 1) // (NS * 8)) * (NS * 8)

    pad = m_pad - m
    src = jnp.concatenate([src, jnp.zeros((pad,), jnp.int32)])
    dst_pad = n + jnp.arange(pad, dtype=jnp.int32) % (n_pad - n)
    dst = jnp.concatenate([dst, dst_pad])
    table = jnp.concatenate([node, node])
    src = src.reshape(NW, cpt * K)
    src = src + (n * (jnp.arange(NW, dtype=jnp.int32) % 2))[:, None]
    src2d = src.reshape(m_pad // K, K)
    dst2d = dst.reshape(m_pad // K, K)

    zeros = jnp.zeros((n_pad, d), jnp.float32)
    acc = _sc_scatter(table, src2d, dst2d, zeros, n_pad, cpt)
    return _tc_combine(node, acc[0, :n], acc[1, :n], jnp.reshape(eps_k, (1,)))
